# baseline (speedup 1.0000x reference)
import functools

import jax
import jax.numpy as jnp
from jax import lax
from jax.experimental import pallas as pl
from jax.experimental.pallas import tpu as pltpu
from jax.experimental.pallas import tpu_sc as plsc

B, C, H, W = 4, 96, 384, 384
BC = B * C
HW = H * W
K = 4096
NC, NS = 2, 16
NW = NC * NS
PPW = BC // NW
CH = 128
NCHUNK = K // CH
LANES = 16


@jax.jit
def _sc_gather(xt, loc):
    mesh = plsc.VectorSubcoreMesh(core_axis_name="c", subcore_axis_name="s")

    @functools.partial(
        pl.kernel,
        out_type=jax.ShapeDtypeStruct((BC, NCHUNK, CH), jnp.float32),
        compiler_params=pltpu.CompilerParams(use_tc_tiling_on_sc=False),
        mesh=mesh,
        scratch_types=[
            pltpu.VMEM((2, K), jnp.int32),
            pltpu.VMEM((NCHUNK, CH), jnp.int32),
            pltpu.VMEM((NCHUNK, CH), jnp.float32),
            pltpu.SemaphoreType.DMA,
        ],
    )
    def k(x_hbm, loc_hbm, out_hbm, loc_v, idx_v, gat_v, gsem):
        wid = lax.axis_index("s") * NC + lax.axis_index("c")
        pltpu.sync_copy(loc_hbm, loc_v)

        def cbody(i, _):
            r = i // (CH // LANES)
            o = pl.multiple_of((i % (CH // LANES)) * LANES, LANES)
            s = pl.multiple_of(i * LANES, LANES)
            v0 = loc_v[0, pl.ds(s, LANES)]
            v1 = loc_v[1, pl.ds(s, LANES)]
            idx_v[r, pl.ds(o, LANES)] = v0 * W + v1
            return ()

        lax.fori_loop(0, K // LANES, cbody, ())

        def pbody(p, _):
            plane = wid * PPW + p
            src_plane = x_hbm.at[plane]

            def fire(r, _):
                pltpu.async_copy(src_plane.at[idx_v.at[r]], gat_v.at[r], gsem)
                return ()

            lax.fori_loop(0, NCHUNK, fire, ())

            def drain(r, _):
                pltpu.make_async_copy(
                    src_plane.at[idx_v.at[r]], gat_v.at[r], gsem
                ).wait()
                return ()

            lax.fori_loop(0, NCHUNK, drain, ())
            pltpu.sync_copy(gat_v, out_hbm.at[plane])
            return ()

        lax.fori_loop(0, PPW, pbody, ())

    return k(xt, loc)


def kernel(x, loc):
    xt = x.reshape(BC, HW)
    out = _sc_gather(xt, loc.astype(jnp.int32))
    return out.reshape(B, C, K)


# speedup vs baseline: 1.0013x; 1.0013x over previous
import functools

import jax
import jax.numpy as jnp
from jax import lax
from jax.experimental import pallas as pl
from jax.experimental.pallas import tpu as pltpu
from jax.experimental.pallas import tpu_sc as plsc

B, C, H, W = 4, 96, 384, 384
BC = B * C
HW = H * W
K = 4096
NC, NS = 2, 16
NW = NC * NS
PPW = BC // NW
CH = 128
NCHUNK = K // CH
LANES = 16


@jax.jit
def _sc_gather(xt, loc):
    mesh = plsc.VectorSubcoreMesh(core_axis_name="c", subcore_axis_name="s")

    @functools.partial(
        pl.kernel,
        out_type=jax.ShapeDtypeStruct((BC, K), jnp.float32),
        compiler_params=pltpu.CompilerParams(use_tc_tiling_on_sc=False),
        mesh=mesh,
        scratch_types=[
            pltpu.VMEM((2, K), jnp.int32),
            pltpu.VMEM((K,), jnp.int32),
            pltpu.VMEM((K,), jnp.float32),
            pltpu.SemaphoreType.DMA,
        ],
    )
    def k(x_hbm, loc_hbm, out_hbm, loc_v, idx_v, gat_v, gsem):
        wid = lax.axis_index("s") * NC + lax.axis_index("c")
        pltpu.sync_copy(loc_hbm, loc_v)

        def cbody(i, _):
            s = pl.multiple_of(i * LANES, LANES)
            v0 = loc_v[0, pl.ds(s, LANES)]
            v1 = loc_v[1, pl.ds(s, LANES)]
            idx_v[pl.ds(s, LANES)] = v0 * W + v1
            return ()

        lax.fori_loop(0, K // LANES, cbody, ())

        def pbody(p, _):
            plane = wid * PPW + p
            src_plane = x_hbm.at[plane]

            pltpu.async_copy(src_plane.at[idx_v], gat_v, gsem)
            pltpu.make_async_copy(src_plane.at[idx_v], gat_v, gsem).wait()
            pltpu.sync_copy(gat_v, out_hbm.at[plane])
            return ()

        lax.fori_loop(0, PPW, pbody, ())

    return k(xt, loc)


def kernel(x, loc):
    xt = x.reshape(BC, HW)
    out = _sc_gather(xt, loc.astype(jnp.int32))
    return out.reshape(B, C, K)


# speedup vs baseline: 1.0351x; 1.0338x over previous
import functools

import jax
import jax.numpy as jnp
from jax import lax
from jax.experimental import pallas as pl
from jax.experimental.pallas import tpu as pltpu
from jax.experimental.pallas import tpu_sc as plsc

B, C, H, W = 4, 96, 384, 384
BC = B * C
HW = H * W
K = 4096
NC, NS = 2, 16
NW = NC * NS
PPW = BC // NW
LANES = 16


@jax.jit
def _sc_gather(xt, loc):
    mesh = plsc.VectorSubcoreMesh(core_axis_name="c", subcore_axis_name="s")

    @functools.partial(
        pl.kernel,
        out_type=jax.ShapeDtypeStruct((BC, K), jnp.float32),
        compiler_params=pltpu.CompilerParams(use_tc_tiling_on_sc=False),
        mesh=mesh,
        scratch_types=[
            pltpu.VMEM((2, K), jnp.int32),
            pltpu.VMEM((K,), jnp.int32),
            pltpu.VMEM((PPW, K), jnp.float32),
            pltpu.SemaphoreType.DMA,
        ],
    )
    def k(x_hbm, loc_hbm, out_hbm, loc_v, idx_v, gat_v, gsem):
        wid = lax.axis_index("s") * NC + lax.axis_index("c")
        pltpu.sync_copy(loc_hbm, loc_v)

        def cbody(i, _):
            s = pl.multiple_of(i * LANES, LANES)
            v0 = loc_v[0, pl.ds(s, LANES)]
            v1 = loc_v[1, pl.ds(s, LANES)]
            idx_v[pl.ds(s, LANES)] = v0 * W + v1
            return ()

        lax.fori_loop(0, K // LANES, cbody, ())

        base = wid * PPW
        for p in range(PPW):
            pltpu.async_copy(x_hbm.at[base + p].at[idx_v], gat_v.at[p], gsem)
        for p in range(PPW):
            pltpu.make_async_copy(x_hbm.at[base + p].at[idx_v], gat_v.at[p],
                                  gsem).wait()
        pltpu.sync_copy(gat_v, out_hbm.at[pl.ds(base, PPW)])

    return k(xt, loc)


def kernel(x, loc):
    xt = x.reshape(BC, HW)
    out = _sc_gather(xt, loc.astype(jnp.int32))
    return out.reshape(B, C, K)
